# baseline (speedup 1.0000x reference)
import jax
import jax.numpy as jnp
from jax.experimental import pallas as pl

_NODE_COUNT = 100


def _agent_mlp_kernel(xg_ref, W1_ref, b1_ref, We_ref, be_ref, out_ref):
    h = jnp.dot(xg_ref[...], W1_ref[...], preferred_element_type=jnp.float32)
    h = jnp.maximum(h + b1_ref[...], 0.0)
    out_ref[...] = (
        jnp.dot(h, We_ref[...], preferred_element_type=jnp.float32) + be_ref[...]
    )


def kernel(x, edge_index, node_count, W1, b1, Wc, bc, We, be):
    N, D = x.shape
    H = W1.shape[1]
    Z = We.shape[1]
    A = (N + _NODE_COUNT - 1) // _NODE_COUNT
    xg = x.reshape(A, _NODE_COUNT * D)
    return pl.pallas_call(
        _agent_mlp_kernel,
        out_shape=jax.ShapeDtypeStruct((A, Z), jnp.float32),
        grid=(1,),
        in_specs=[
            pl.BlockSpec((A, D), lambda i: (0, 0)),
            pl.BlockSpec((D, H), lambda i: (0, 0)),
            pl.BlockSpec((1, H), lambda i: (0, 0)),
            pl.BlockSpec((H, Z), lambda i: (0, 0)),
            pl.BlockSpec((1, Z), lambda i: (0, 0)),
        ],
        out_specs=pl.BlockSpec((A, Z), lambda i: (0, 0)),
    )(xg, W1, b1.reshape(1, H), We, be.reshape(1, Z))


# speedup vs baseline: 1.0082x; 1.0082x over previous
import jax
import jax.numpy as jnp
from jax.experimental import pallas as pl

_NODE_COUNT = 100


_SPLIT = 8


def _agent_mlp_kernel(*refs):
    xg_refs = refs[:_SPLIT]
    W1_ref, b1_ref, We_ref, be_ref, out_ref = refs[_SPLIT:]
    A = out_ref.shape[0]
    xg = jnp.concatenate([r[...] for r in xg_refs], axis=0)
    h = jnp.dot(xg, W1_ref[...], preferred_element_type=jnp.float32)
    h = jnp.maximum(h + b1_ref[...], 0.0)
    out = jnp.dot(h, We_ref[...], preferred_element_type=jnp.float32) + be_ref[...]
    out_ref[...] = out[:A]


def _make_x_spec(j, rows, D):
    return pl.BlockSpec((rows, D), lambda i, j=j: (j, 0))


def kernel(x, edge_index, node_count, W1, b1, Wc, bc, We, be):
    N, D = x.shape
    H = W1.shape[1]
    Z = We.shape[1]
    A = (N + _NODE_COUNT - 1) // _NODE_COUNT
    xg = x.reshape(A, _NODE_COUNT * D)
    rows = -(-A // _SPLIT)
    rows = -(-rows // 8) * 8
    return pl.pallas_call(
        _agent_mlp_kernel,
        out_shape=jax.ShapeDtypeStruct((A, Z), jnp.float32),
        grid=(1,),
        in_specs=[_make_x_spec(j, rows, D) for j in range(_SPLIT)]
        + [
            pl.BlockSpec((D, H), lambda i: (0, 0)),
            pl.BlockSpec((1, H), lambda i: (0, 0)),
            pl.BlockSpec((H, Z), lambda i: (0, 0)),
            pl.BlockSpec((1, Z), lambda i: (0, 0)),
        ],
        out_specs=pl.BlockSpec((A, Z), lambda i: (0, 0)),
    )(*([xg] * _SPLIT), W1, b1.reshape(1, H), We, be.reshape(1, Z))


# speedup vs baseline: 3.5689x; 3.5400x over previous
import jax
import jax.numpy as jnp
from jax.experimental import pallas as pl
from jax.experimental.pallas import tpu as pltpu

_NODE_COUNT = 100


def _agent_mlp_kernel(x_hbm, W1_ref, b1_ref, We_ref, be_ref, out_ref, xs, sem):
    A = out_ref.shape[0]
    copies = [
        pltpu.make_async_copy(x_hbm.at[a * _NODE_COUNT], xs.at[a], sem)
        for a in range(A)
    ]
    for cp in copies:
        cp.start()
    for cp in copies:
        cp.wait()
    h = jnp.dot(xs[...], W1_ref[...], preferred_element_type=jnp.float32)
    h = jnp.maximum(h + b1_ref[...], 0.0)
    out = jnp.dot(h, We_ref[...], preferred_element_type=jnp.float32) + be_ref[...]
    out_ref[...] = out[:A]


def kernel(x, edge_index, node_count, W1, b1, Wc, bc, We, be):
    N, D = x.shape
    H = W1.shape[1]
    Z = We.shape[1]
    A = (N + _NODE_COUNT - 1) // _NODE_COUNT
    A_pad = -(-A // 8) * 8
    return pl.pallas_call(
        _agent_mlp_kernel,
        out_shape=jax.ShapeDtypeStruct((A, Z), jnp.float32),
        grid=(1,),
        in_specs=[
            pl.BlockSpec(memory_space=pl.ANY),
            pl.BlockSpec((D, H), lambda i: (0, 0)),
            pl.BlockSpec((1, H), lambda i: (0, 0)),
            pl.BlockSpec((H, Z), lambda i: (0, 0)),
            pl.BlockSpec((1, Z), lambda i: (0, 0)),
        ],
        out_specs=pl.BlockSpec((A, Z), lambda i: (0, 0)),
        scratch_shapes=[
            pltpu.VMEM((A_pad, D), jnp.float32),
            pltpu.SemaphoreType.DMA,
        ],
    )(x, W1, b1.reshape(1, H), We, be.reshape(1, Z))
